# speedup vs baseline: 20.9314x; 20.9314x over previous
import functools

import jax
import jax.numpy as jnp
from jax import lax
from jax.experimental import pallas as pl
from jax.experimental.pallas import tpu as pltpu
from jax.experimental.pallas import tpu_sc as plsc

N = 10000
E = 320000
D = 128
NC, NS = 2, 16
NW = NC * NS
EPW = E // NW
CH = 128
FC = EPW // CH
TAIL = EPW - FC * CH
NPAD = 10240
RPT = NPAD // NS
DEGW = 16
BN = 400

_MESH = plsc.VectorSubcoreMesh(core_axis_name="c", subcore_axis_name="s")


def _deg_body(src_hbm, deg_out, idx_v, idx_t, ones_v, zero_v, deg_sh):
    cid = lax.axis_index("c")
    tid = lax.axis_index("s")
    wid = cid * NS + tid
    base = wid * EPW

    def fill(i, carry):
        ones_v[i, :] = jnp.ones((DEGW,), jnp.float32)
        zero_v[i, :] = jnp.zeros((DEGW,), jnp.float32)
        return carry

    lax.fori_loop(0, CH, fill, 0)

    r0 = tid * RPT

    def zero(i, carry):
        pltpu.sync_copy(zero_v, deg_sh.at[pl.ds(r0 + i * CH, CH)])
        return carry

    lax.fori_loop(0, RPT // CH, zero, 0)
    plsc.subcore_barrier()

    def body(c, carry):
        pltpu.sync_copy(src_hbm.at[pl.ds(base + c * CH, CH)], idx_v)
        pltpu.sync_copy(ones_v, deg_sh.at[idx_v], add=True)
        return carry

    lax.fori_loop(0, FC, body, 0)
    pltpu.sync_copy(src_hbm.at[pl.ds(base + FC * CH, TAIL)], idx_t)
    pltpu.sync_copy(ones_v.at[pl.ds(0, TAIL)], deg_sh.at[idx_t], add=True)
    plsc.subcore_barrier()
    pltpu.sync_copy(deg_sh.at[pl.ds(r0, RPT)], deg_out.at[cid, pl.ds(r0, RPT)])


_deg_call = pl.kernel(
    _deg_body,
    out_type=jax.ShapeDtypeStruct((NC, NPAD, DEGW), jnp.float32),
    mesh=_MESH,
    scratch_types=[
        pltpu.VMEM((CH,), jnp.int32),
        pltpu.VMEM((TAIL,), jnp.int32),
        pltpu.VMEM((CH, DEGW), jnp.float32),
        pltpu.VMEM((CH, DEGW), jnp.float32),
        pltpu.VMEM_SHARED((NPAD, DEGW), jnp.float32),
    ],
)


def _edge_body(g_hbm, src_hbm, dst_hbm, acc_out,
               sidx, didx, idx_t, rows, rows_t, acc_sh, sem):
    cid = lax.axis_index("c")
    tid = lax.axis_index("s")
    wid = cid * NS + tid
    base = wid * EPW

    def fillz(i, carry):
        for j in range(D // 16):
            rows[i, pl.ds(j * 16, 16)] = jnp.zeros((16,), jnp.float32)
        return carry

    lax.fori_loop(0, CH, fillz, 0)

    r0 = tid * RPT

    def zero(i, carry):
        pltpu.sync_copy(rows, acc_sh.at[pl.ds(r0 + i * CH, CH)])
        return carry

    lax.fori_loop(0, RPT // CH, zero, 0)
    plsc.subcore_barrier()

    def body(c, carry):
        o = base + c * CH
        pltpu.sync_copy(src_hbm.at[pl.ds(o, CH)], sidx)
        pltpu.sync_copy(dst_hbm.at[pl.ds(o, CH)], didx)
        pltpu.async_copy(g_hbm.at[sidx], rows, sem).wait()
        pltpu.sync_copy(rows, acc_sh.at[didx], add=True)
        return carry

    lax.fori_loop(0, FC, body, 0)
    o = base + FC * CH
    pltpu.sync_copy(src_hbm.at[pl.ds(o, TAIL)], idx_t)
    pltpu.async_copy(g_hbm.at[idx_t], rows_t, sem).wait()
    pltpu.sync_copy(dst_hbm.at[pl.ds(o, TAIL)], idx_t)
    pltpu.sync_copy(rows_t, acc_sh.at[idx_t], add=True)
    plsc.subcore_barrier()
    pltpu.sync_copy(acc_sh.at[pl.ds(r0, RPT)], acc_out.at[cid, pl.ds(r0, RPT)])


_edge_call = pl.kernel(
    _edge_body,
    out_type=jax.ShapeDtypeStruct((NC, NPAD, D), jnp.float32),
    mesh=_MESH,
    scratch_types=[
        pltpu.VMEM((CH,), jnp.int32),
        pltpu.VMEM((CH,), jnp.int32),
        pltpu.VMEM((TAIL,), jnp.int32),
        pltpu.VMEM((CH, D), jnp.float32),
        pltpu.VMEM((TAIL, D), jnp.float32),
        pltpu.VMEM_SHARED((NPAD, D), jnp.float32),
        pltpu.SemaphoreType.DMA,
    ],
)


def _dense_body(x_ref, w_ref, deg_ref, g_ref):
    deg01 = deg_ref[0] + deg_ref[1]
    dis = lax.rsqrt(deg01[:, 0:1] + 1.0)
    h = jnp.dot(x_ref[:], w_ref[:], preferred_element_type=jnp.float32)
    g_ref[:] = dis * h


def _dense_call(x, w, degp):
    return pl.pallas_call(
        _dense_body,
        grid=(N // BN,),
        in_specs=[
            pl.BlockSpec((BN, D), lambda i: (i, 0)),
            pl.BlockSpec((D, D), lambda i: (0, 0)),
            pl.BlockSpec((NC, BN, DEGW), lambda i: (0, i, 0)),
        ],
        out_specs=pl.BlockSpec((BN, D), lambda i: (i, 0)),
        out_shape=jax.ShapeDtypeStruct((N, D), jnp.float32),
    )(x, w, degp)


def _final_body(acc_ref, g_ref, deg_ref, out_ref):
    deg01 = deg_ref[0] + deg_ref[1]
    dis = lax.rsqrt(deg01[:, 0:1] + 1.0)
    out_ref[:] = dis * (acc_ref[0] + acc_ref[1] + g_ref[:])


def _final_call(acc, g, degp):
    return pl.pallas_call(
        _final_body,
        grid=(N // BN,),
        in_specs=[
            pl.BlockSpec((NC, BN, D), lambda i: (0, i, 0)),
            pl.BlockSpec((BN, D), lambda i: (i, 0)),
            pl.BlockSpec((NC, BN, DEGW), lambda i: (0, i, 0)),
        ],
        out_specs=pl.BlockSpec((BN, D), lambda i: (i, 0)),
        out_shape=jax.ShapeDtypeStruct((N, D), jnp.float32),
    )(acc, g, degp)


def kernel(x, edge_index, W):
    src = edge_index[0].astype(jnp.int32)
    dst = edge_index[1].astype(jnp.int32)
    degp = _deg_call(src)
    g = _dense_call(x, W, degp)
    acc = _edge_call(g, src, dst)
    return _final_call(acc, g, degp)


# speedup vs baseline: 29.3219x; 1.4009x over previous
import jax
import jax.numpy as jnp
from jax import lax
from jax.experimental import pallas as pl
from jax.experimental.pallas import tpu as pltpu
from jax.experimental.pallas import tpu_sc as plsc

N = 10000
E = 320000
D = 128
NC, NS = 2, 16
NW = NC * NS
EPW = E // NW
CH = 128
FC = EPW // CH
TAIL = EPW - FC * CH
NPAD = 10240
RPT = NPAD // NS
DEGW = 16
BN = 400

_MESH = plsc.VectorSubcoreMesh(core_axis_name="c", subcore_axis_name="s")


def _deg_body(src_hbm, deg_out, idx0, idx1, idx_t, ones_v, zero_v, deg_sh,
              sem0, sem1):
    cid = lax.axis_index("c")
    tid = lax.axis_index("s")
    wid = cid * NS + tid
    base = wid * EPW
    idx = (idx0, idx1)
    sem = (sem0, sem1)

    def fill(i, carry):
        ones_v[i, :] = jnp.ones((DEGW,), jnp.float32)
        zero_v[i, :] = jnp.zeros((DEGW,), jnp.float32)
        return carry

    lax.fori_loop(0, CH, fill, 0)

    r0 = tid * RPT

    def zero(i, carry):
        pltpu.sync_copy(zero_v, deg_sh.at[pl.ds(r0 + i * CH, CH)])
        return carry

    lax.fori_loop(0, RPT // CH, zero, 0)
    plsc.subcore_barrier()

    pltpu.sync_copy(src_hbm.at[pl.ds(base, CH)], idx0)
    pltpu.async_copy(ones_v, deg_sh.at[idx0], sem0, add=True)

    def visits(k, carry):
        for b in (1, 0):
            c = (1 + 2 * k) if b == 1 else (2 + 2 * k)
            if b == 1:
                @pl.when(k >= 1)
                def _():
                    pltpu.make_async_copy(ones_v, deg_sh.at[idx[b]],
                                          sem[b]).wait()
            else:
                pltpu.make_async_copy(ones_v, deg_sh.at[idx[b]], sem[b]).wait()
            pltpu.sync_copy(src_hbm.at[pl.ds(base + c * CH, CH)], idx[b])
            pltpu.async_copy(ones_v, deg_sh.at[idx[b]], sem[b], add=True)
        return carry

    lax.fori_loop(0, 38, visits, 0)
    pltpu.make_async_copy(ones_v, deg_sh.at[idx1], sem1).wait()
    pltpu.sync_copy(src_hbm.at[pl.ds(base + 77 * CH, CH)], idx1)
    pltpu.async_copy(ones_v, deg_sh.at[idx1], sem1, add=True)
    pltpu.sync_copy(src_hbm.at[pl.ds(base + FC * CH, TAIL)], idx_t)
    pltpu.sync_copy(ones_v.at[pl.ds(0, TAIL)], deg_sh.at[idx_t], add=True)
    pltpu.make_async_copy(ones_v, deg_sh.at[idx0], sem0).wait()
    pltpu.make_async_copy(ones_v, deg_sh.at[idx1], sem1).wait()
    plsc.subcore_barrier()
    pltpu.sync_copy(deg_sh.at[pl.ds(r0, RPT)], deg_out.at[cid, pl.ds(r0, RPT)])


_deg_call = pl.kernel(
    _deg_body,
    out_type=jax.ShapeDtypeStruct((NC, NPAD, DEGW), jnp.float32),
    mesh=_MESH,
    scratch_types=[
        pltpu.VMEM((CH,), jnp.int32),
        pltpu.VMEM((CH,), jnp.int32),
        pltpu.VMEM((TAIL,), jnp.int32),
        pltpu.VMEM((CH, DEGW), jnp.float32),
        pltpu.VMEM((CH, DEGW), jnp.float32),
        pltpu.VMEM_SHARED((NPAD, DEGW), jnp.float32),
        pltpu.SemaphoreType.DMA,
        pltpu.SemaphoreType.DMA,
    ],
)


def _edge_body(g_hbm, src_hbm, dst_hbm, acc_out,
               sidx0, didx0, rows0, sidx1, didx1, rows1,
               idx_t, rows_t, acc_sh, gsem0, gsem1, ssem0, ssem1, tsem):
    cid = lax.axis_index("c")
    tid = lax.axis_index("s")
    wid = cid * NS + tid
    base = wid * EPW
    sidx = (sidx0, sidx1)
    didx = (didx0, didx1)
    rows = (rows0, rows1)
    gsem = (gsem0, gsem1)
    ssem = (ssem0, ssem1)

    def fillz(i, carry):
        for j in range(D // 16):
            rows0[i, pl.ds(j * 16, 16)] = jnp.zeros((16,), jnp.float32)
        return carry

    lax.fori_loop(0, CH, fillz, 0)

    r0 = tid * RPT

    def zero(i, carry):
        pltpu.sync_copy(rows0, acc_sh.at[pl.ds(r0 + i * CH, CH)])
        return carry

    lax.fori_loop(0, RPT // CH, zero, 0)
    plsc.subcore_barrier()

    def load_and_gather(c, b):
        o = base + c * CH
        pltpu.sync_copy(src_hbm.at[pl.ds(o, CH)], sidx[b])
        pltpu.sync_copy(dst_hbm.at[pl.ds(o, CH)], didx[b])
        pltpu.async_copy(g_hbm.at[sidx[b]], rows[b], gsem[b])

    def wait_gather(b):
        pltpu.make_async_copy(g_hbm.at[sidx[b]], rows[b], gsem[b]).wait()

    def start_scatter(b):
        pltpu.async_copy(rows[b], acc_sh.at[didx[b]], ssem[b], add=True)

    def wait_scatter(b):
        pltpu.make_async_copy(rows[b], acc_sh.at[didx[b]], ssem[b]).wait()

    load_and_gather(0, 0)

    def visits(k, carry):
        for b in (1, 0):
            c = (1 + 2 * k) if b == 1 else (2 + 2 * k)
            if b == 1:
                @pl.when(k >= 1)
                def _():
                    wait_scatter(b)
            else:
                wait_scatter(b)
            load_and_gather(c, b)
            wait_gather(1 - b)
            start_scatter(1 - b)
        return carry

    lax.fori_loop(0, 38, visits, 0)
    wait_scatter(1)
    load_and_gather(77, 1)
    wait_gather(0)
    start_scatter(0)
    wait_gather(1)
    start_scatter(1)
    o = base + FC * CH
    pltpu.sync_copy(src_hbm.at[pl.ds(o, TAIL)], idx_t)
    pltpu.async_copy(g_hbm.at[idx_t], rows_t, tsem).wait()
    pltpu.sync_copy(dst_hbm.at[pl.ds(o, TAIL)], idx_t)
    pltpu.sync_copy(rows_t, acc_sh.at[idx_t], add=True)
    wait_scatter(0)
    wait_scatter(1)
    plsc.subcore_barrier()
    pltpu.sync_copy(acc_sh.at[pl.ds(r0, RPT)], acc_out.at[cid, pl.ds(r0, RPT)])


_edge_call = pl.kernel(
    _edge_body,
    out_type=jax.ShapeDtypeStruct((NC, NPAD, D), jnp.float32),
    mesh=_MESH,
    scratch_types=[
        pltpu.VMEM((CH,), jnp.int32),
        pltpu.VMEM((CH,), jnp.int32),
        pltpu.VMEM((CH, D), jnp.float32),
        pltpu.VMEM((CH,), jnp.int32),
        pltpu.VMEM((CH,), jnp.int32),
        pltpu.VMEM((CH, D), jnp.float32),
        pltpu.VMEM((TAIL,), jnp.int32),
        pltpu.VMEM((TAIL, D), jnp.float32),
        pltpu.VMEM_SHARED((NPAD, D), jnp.float32),
        pltpu.SemaphoreType.DMA,
        pltpu.SemaphoreType.DMA,
        pltpu.SemaphoreType.DMA,
        pltpu.SemaphoreType.DMA,
        pltpu.SemaphoreType.DMA,
    ],
)


def _mm_body(x_ref, w_ref, h_ref):
    h_ref[:] = jnp.dot(x_ref[:], w_ref[:], preferred_element_type=jnp.float32)


def _mm_call(x, w):
    return pl.pallas_call(
        _mm_body,
        grid=(N // BN,),
        in_specs=[
            pl.BlockSpec((BN, D), lambda i: (i, 0)),
            pl.BlockSpec((D, D), lambda i: (0, 0)),
        ],
        out_specs=pl.BlockSpec((BN, D), lambda i: (i, 0)),
        out_shape=jax.ShapeDtypeStruct((N, D), jnp.float32),
    )(x, w)


def _scale_body(h_ref, deg_ref, g_ref):
    deg01 = deg_ref[0] + deg_ref[1]
    dis = lax.rsqrt(deg01[:, 0:1] + 1.0)
    g_ref[:] = dis * h_ref[:]


def _scale_call(h, degp):
    return pl.pallas_call(
        _scale_body,
        grid=(N // BN,),
        in_specs=[
            pl.BlockSpec((BN, D), lambda i: (i, 0)),
            pl.BlockSpec((NC, BN, DEGW), lambda i: (0, i, 0)),
        ],
        out_specs=pl.BlockSpec((BN, D), lambda i: (i, 0)),
        out_shape=jax.ShapeDtypeStruct((N, D), jnp.float32),
    )(h, degp)


def _final_body(acc_ref, g_ref, deg_ref, out_ref):
    deg01 = deg_ref[0] + deg_ref[1]
    dis = lax.rsqrt(deg01[:, 0:1] + 1.0)
    out_ref[:] = dis * (acc_ref[0] + acc_ref[1] + g_ref[:])


def _final_call(acc, g, degp):
    return pl.pallas_call(
        _final_body,
        grid=(N // BN,),
        in_specs=[
            pl.BlockSpec((NC, BN, D), lambda i: (0, i, 0)),
            pl.BlockSpec((BN, D), lambda i: (i, 0)),
            pl.BlockSpec((NC, BN, DEGW), lambda i: (0, i, 0)),
        ],
        out_specs=pl.BlockSpec((BN, D), lambda i: (i, 0)),
        out_shape=jax.ShapeDtypeStruct((N, D), jnp.float32),
    )(acc, g, degp)


def kernel(x, edge_index, W):
    src = edge_index[0].astype(jnp.int32)
    dst = edge_index[1].astype(jnp.int32)
    h = _mm_call(x, W)
    degp = _deg_call(src)
    g = _scale_call(h, degp)
    acc = _edge_call(g, src, dst)
    return _final_call(acc, g, degp)


# speedup vs baseline: 37.5023x; 1.2790x over previous
import jax
import jax.numpy as jnp
from jax import lax
from jax.experimental import pallas as pl
from jax.experimental.pallas import tpu as pltpu
from jax.experimental.pallas import tpu_sc as plsc

N = 10000
E = 320000
D = 128
NC, NS = 2, 16
NW = NC * NS
EPW = E // NW
CH = 128
FC = EPW // CH
TAIL = EPW - FC * CH
CHE = 64
FCE = EPW // CHE
NPAD = 10240
RPT = NPAD // NS
DEGW = 16
BN = 400
NB = 4

_MESH = plsc.VectorSubcoreMesh(core_axis_name="c", subcore_axis_name="s")


def _deg_body(src_hbm, deg_out,
              idx0, idx1, idx2, idx3, idx_t, ones_v, zero_v, deg_sh,
              lsem0, lsem1, lsem2, lsem3, ssem0, ssem1, ssem2, ssem3):
    cid = lax.axis_index("c")
    tid = lax.axis_index("s")
    wid = cid * NS + tid
    base = wid * EPW
    idx = (idx0, idx1, idx2, idx3)
    lsem = (lsem0, lsem1, lsem2, lsem3)
    ssem = (ssem0, ssem1, ssem2, ssem3)

    def fill(i, carry):
        ones_v[i, :] = jnp.ones((DEGW,), jnp.float32)
        zero_v[i, :] = jnp.zeros((DEGW,), jnp.float32)
        return carry

    lax.fori_loop(0, CH, fill, 0)

    r0 = tid * RPT

    def zero(i, carry):
        pltpu.sync_copy(zero_v, deg_sh.at[pl.ds(r0 + i * CH, CH)])
        return carry

    lax.fori_loop(0, RPT // CH, zero, 0)
    plsc.subcore_barrier()

    def l_start(c, b):
        pltpu.async_copy(src_hbm.at[pl.ds(base + c * CH, CH)], idx[b], lsem[b])

    def l_wait(b):
        pltpu.make_async_copy(src_hbm.at[pl.ds(base, CH)], idx[b],
                              lsem[b]).wait()

    def s_start(b):
        pltpu.async_copy(ones_v, deg_sh.at[idx[b]], ssem[b], add=True)

    def s_wait(b):
        pltpu.make_async_copy(ones_v, deg_sh.at[idx[b]], ssem[b]).wait()

    l_start(0, 0)

    def visits(k, carry):
        for p in range(NB):
            bn = (p + 1) % NB
            if p == 3:
                s_wait(bn)
            else:
                @pl.when(k >= 1)
                def _():
                    s_wait(bn)
            l_start(4 * k + p + 1, bn)
            l_wait(p)
            s_start(p)
        return carry

    lax.fori_loop(0, 19, visits, 0)
    s_wait(1)
    l_start(77, 1)
    l_wait(0)
    s_start(0)
    s_wait(2)
    l_wait(1)
    s_start(1)
    pltpu.sync_copy(src_hbm.at[pl.ds(base + FC * CH, TAIL)], idx_t)
    pltpu.sync_copy(ones_v.at[pl.ds(0, TAIL)], deg_sh.at[idx_t], add=True)
    s_wait(3)
    s_wait(0)
    s_wait(1)
    plsc.subcore_barrier()
    pltpu.sync_copy(deg_sh.at[pl.ds(r0, RPT)], deg_out.at[cid, pl.ds(r0, RPT)])


_deg_call = pl.kernel(
    _deg_body,
    out_type=jax.ShapeDtypeStruct((NC, NPAD, DEGW), jnp.float32),
    mesh=_MESH,
    scratch_types=(
        [pltpu.VMEM((CH,), jnp.int32)] * NB
        + [
            pltpu.VMEM((TAIL,), jnp.int32),
            pltpu.VMEM((CH, DEGW), jnp.float32),
            pltpu.VMEM((CH, DEGW), jnp.float32),
            pltpu.VMEM_SHARED((NPAD, DEGW), jnp.float32),
        ]
        + [pltpu.SemaphoreType.DMA] * (2 * NB)
    ),
)


def _edge_body(g_hbm, src_hbm, dst_hbm, acc_out,
               sidx0, sidx1, sidx2, sidx3, didx0, didx1, didx2, didx3,
               rows0, rows1, rows2, rows3, idx_t, rows_t, acc_sh,
               lsem0, lsem1, lsem2, lsem3, gsem0, gsem1, gsem2, gsem3,
               ssem0, ssem1, ssem2, ssem3, tsem):
    cid = lax.axis_index("c")
    tid = lax.axis_index("s")
    wid = cid * NS + tid
    base = wid * EPW
    sidx = (sidx0, sidx1, sidx2, sidx3)
    didx = (didx0, didx1, didx2, didx3)
    rows = (rows0, rows1, rows2, rows3)
    lsem = (lsem0, lsem1, lsem2, lsem3)
    gsem = (gsem0, gsem1, gsem2, gsem3)
    ssem = (ssem0, ssem1, ssem2, ssem3)

    def fillz(i, carry):
        for j in range(D // 16):
            rows0[i, pl.ds(j * 16, 16)] = jnp.zeros((16,), jnp.float32)
        return carry

    lax.fori_loop(0, CHE, fillz, 0)

    r0 = tid * RPT

    def zero(i, carry):
        pltpu.sync_copy(rows0, acc_sh.at[pl.ds(r0 + i * CHE, CHE)])
        return carry

    lax.fori_loop(0, RPT // CHE, zero, 0)
    plsc.subcore_barrier()

    def l_start(c, b):
        o = base + c * CHE
        pltpu.async_copy(src_hbm.at[pl.ds(o, CHE)], sidx[b], lsem[b])
        pltpu.async_copy(dst_hbm.at[pl.ds(o, CHE)], didx[b], lsem[b])

    def l_wait(b):
        pltpu.make_async_copy(src_hbm.at[pl.ds(base, CHE)], sidx[b],
                              lsem[b]).wait()
        pltpu.make_async_copy(dst_hbm.at[pl.ds(base, CHE)], didx[b],
                              lsem[b]).wait()

    def g_start(b):
        pltpu.async_copy(g_hbm.at[sidx[b]], rows[b], gsem[b])

    def g_wait(b):
        pltpu.make_async_copy(g_hbm.at[sidx[b]], rows[b], gsem[b]).wait()

    def s_start(b):
        pltpu.async_copy(rows[b], acc_sh.at[didx[b]], ssem[b], add=True)

    def s_wait(b):
        pltpu.make_async_copy(rows[b], acc_sh.at[didx[b]], ssem[b]).wait()

    l_start(0, 0)
    l_start(1, 1)
    l_wait(0)
    g_start(0)

    def visits(k, carry):
        for p in range(NB):
            bc = (1 + p) % NB
            bn = (2 + p) % NB
            bp = p
            if p >= 2:
                s_wait(bn)
            else:
                @pl.when(k >= 1)
                def _():
                    s_wait(bn)
            l_start(1 + 4 * k + p + 1, bn)
            l_wait(bc)
            g_start(bc)
            g_wait(bp)
            s_start(bp)
        return carry

    lax.fori_loop(0, 38, visits, 0)
    s_wait(2)
    l_start(154, 2)
    l_wait(1)
    g_start(1)
    g_wait(0)
    s_start(0)
    s_wait(3)
    l_start(155, 3)
    l_wait(2)
    g_start(2)
    g_wait(1)
    s_start(1)
    s_wait(0)
    l_wait(3)
    g_start(3)
    g_wait(2)
    s_start(2)
    g_wait(3)
    s_start(3)
    o = base + FCE * CHE
    pltpu.sync_copy(src_hbm.at[pl.ds(o, TAIL)], idx_t)
    pltpu.async_copy(g_hbm.at[idx_t], rows_t, tsem).wait()
    pltpu.sync_copy(dst_hbm.at[pl.ds(o, TAIL)], idx_t)
    pltpu.sync_copy(rows_t, acc_sh.at[idx_t], add=True)
    s_wait(1)
    s_wait(2)
    s_wait(3)
    plsc.subcore_barrier()
    pltpu.sync_copy(acc_sh.at[pl.ds(r0, RPT)], acc_out.at[cid, pl.ds(r0, RPT)])


_edge_call = pl.kernel(
    _edge_body,
    out_type=jax.ShapeDtypeStruct((NC, NPAD, D), jnp.float32),
    mesh=_MESH,
    scratch_types=(
        [pltpu.VMEM((CHE,), jnp.int32)] * (2 * NB)
        + [pltpu.VMEM((CHE, D), jnp.float32)] * NB
        + [
            pltpu.VMEM((TAIL,), jnp.int32),
            pltpu.VMEM((TAIL, D), jnp.float32),
            pltpu.VMEM_SHARED((NPAD, D), jnp.float32),
        ]
        + [pltpu.SemaphoreType.DMA] * (3 * NB + 1)
    ),
)


def _mm_body(x_ref, w_ref, h_ref):
    h_ref[:] = jnp.dot(x_ref[:], w_ref[:], preferred_element_type=jnp.float32)


def _mm_call(x, w):
    return pl.pallas_call(
        _mm_body,
        grid=(N // BN,),
        in_specs=[
            pl.BlockSpec((BN, D), lambda i: (i, 0)),
            pl.BlockSpec((D, D), lambda i: (0, 0)),
        ],
        out_specs=pl.BlockSpec((BN, D), lambda i: (i, 0)),
        out_shape=jax.ShapeDtypeStruct((N, D), jnp.float32),
    )(x, w)


def _scale_body(h_ref, deg_ref, g_ref):
    deg01 = deg_ref[0] + deg_ref[1]
    dis = lax.rsqrt(deg01[:, 0:1] + 1.0)
    g_ref[:] = dis * h_ref[:]


def _scale_call(h, degp):
    return pl.pallas_call(
        _scale_body,
        grid=(N // BN,),
        in_specs=[
            pl.BlockSpec((BN, D), lambda i: (i, 0)),
            pl.BlockSpec((NC, BN, DEGW), lambda i: (0, i, 0)),
        ],
        out_specs=pl.BlockSpec((BN, D), lambda i: (i, 0)),
        out_shape=jax.ShapeDtypeStruct((N, D), jnp.float32),
    )(h, degp)


def _final_body(acc_ref, g_ref, deg_ref, out_ref):
    deg01 = deg_ref[0] + deg_ref[1]
    dis = lax.rsqrt(deg01[:, 0:1] + 1.0)
    out_ref[:] = dis * (acc_ref[0] + acc_ref[1] + g_ref[:])


def _final_call(acc, g, degp):
    return pl.pallas_call(
        _final_body,
        grid=(N // BN,),
        in_specs=[
            pl.BlockSpec((NC, BN, D), lambda i: (0, i, 0)),
            pl.BlockSpec((BN, D), lambda i: (i, 0)),
            pl.BlockSpec((NC, BN, DEGW), lambda i: (0, i, 0)),
        ],
        out_specs=pl.BlockSpec((BN, D), lambda i: (i, 0)),
        out_shape=jax.ShapeDtypeStruct((N, D), jnp.float32),
    )(acc, g, degp)


def kernel(x, edge_index, W):
    src = edge_index[0].astype(jnp.int32)
    dst = edge_index[1].astype(jnp.int32)
    h = _mm_call(x, W)
    degp = _deg_call(src)
    g = _scale_call(h, degp)
    acc = _edge_call(g, src, dst)
    return _final_call(acc, g, degp)


# speedup vs baseline: 37.7779x; 1.0073x over previous
import jax
import jax.numpy as jnp
from jax import lax
from jax.experimental import pallas as pl
from jax.experimental.pallas import tpu as pltpu
from jax.experimental.pallas import tpu_sc as plsc

N = 10000
E = 320000
D = 128
NC, NS = 2, 16
NW = NC * NS
EPW = E // NW
CH = 128
FC = EPW // CH
TAIL = EPW - FC * CH
CHE = 64
FCE = EPW // CHE
NPAD = 10240
RPT = NPAD // NS
DEGW = 16
BN = 400
NB = 4

_MESH = plsc.VectorSubcoreMesh(core_axis_name="c", subcore_axis_name="s")


def _deg_body(src_hbm, deg_out,
              idx0, idx1, idx2, idx3, idx_t, ones_v, zero_v, deg_sh,
              lsem0, lsem1, lsem2, lsem3, ssem0, ssem1, ssem2, ssem3):
    cid = lax.axis_index("c")
    tid = lax.axis_index("s")
    wid = cid * NS + tid
    base = wid * EPW
    idx = (idx0, idx1, idx2, idx3)
    lsem = (lsem0, lsem1, lsem2, lsem3)
    ssem = (ssem0, ssem1, ssem2, ssem3)

    def l_start(c, b):
        pltpu.async_copy(src_hbm.at[pl.ds(base + c * CH, CH)], idx[b], lsem[b])

    def l_wait(b):
        pltpu.make_async_copy(src_hbm.at[pl.ds(base, CH)], idx[b],
                              lsem[b]).wait()

    def s_start(b):
        pltpu.async_copy(ones_v, deg_sh.at[idx[b]], ssem[b], add=True)

    def s_wait(b):
        pltpu.make_async_copy(ones_v, deg_sh.at[idx[b]], ssem[b]).wait()

    l_start(0, 0)

    def fill(i, carry):
        ones_v[i, :] = jnp.ones((DEGW,), jnp.float32)
        zero_v[i, :] = jnp.zeros((DEGW,), jnp.float32)
        return carry

    lax.fori_loop(0, CH, fill, 0)

    r0 = tid * RPT

    def zero(i, carry):
        pltpu.sync_copy(zero_v, deg_sh.at[pl.ds(r0 + i * CH, CH)])
        return carry

    lax.fori_loop(0, RPT // CH, zero, 0)
    plsc.subcore_barrier()


    def visits(k, carry):
        for p in range(NB):
            bn = (p + 1) % NB
            if p == 3:
                s_wait(bn)
            else:
                @pl.when(k >= 1)
                def _():
                    s_wait(bn)
            l_start(4 * k + p + 1, bn)
            l_wait(p)
            s_start(p)
        return carry

    lax.fori_loop(0, 19, visits, 0)
    s_wait(1)
    l_start(77, 1)
    l_wait(0)
    s_start(0)
    s_wait(2)
    l_wait(1)
    s_start(1)
    pltpu.sync_copy(src_hbm.at[pl.ds(base + FC * CH, TAIL)], idx_t)
    pltpu.sync_copy(ones_v.at[pl.ds(0, TAIL)], deg_sh.at[idx_t], add=True)
    s_wait(3)
    s_wait(0)
    s_wait(1)
    plsc.subcore_barrier()
    pltpu.sync_copy(deg_sh.at[pl.ds(r0, RPT)], deg_out.at[cid, pl.ds(r0, RPT)])


_deg_call = pl.kernel(
    _deg_body,
    out_type=jax.ShapeDtypeStruct((NC, NPAD, DEGW), jnp.float32),
    mesh=_MESH,
    scratch_types=(
        [pltpu.VMEM((CH,), jnp.int32)] * NB
        + [
            pltpu.VMEM((TAIL,), jnp.int32),
            pltpu.VMEM((CH, DEGW), jnp.float32),
            pltpu.VMEM((CH, DEGW), jnp.float32),
            pltpu.VMEM_SHARED((NPAD, DEGW), jnp.float32),
        ]
        + [pltpu.SemaphoreType.DMA] * (2 * NB)
    ),
)


def _edge_body(g_hbm, src_hbm, dst_hbm, acc_out,
               sidx0, sidx1, sidx2, sidx3, didx0, didx1, didx2, didx3,
               rows0, rows1, rows2, rows3, idx_t, rows_t, acc_sh,
               lsem0, lsem1, lsem2, lsem3, gsem0, gsem1, gsem2, gsem3,
               ssem0, ssem1, ssem2, ssem3, tsem):
    cid = lax.axis_index("c")
    tid = lax.axis_index("s")
    wid = cid * NS + tid
    base = wid * EPW
    sidx = (sidx0, sidx1, sidx2, sidx3)
    didx = (didx0, didx1, didx2, didx3)
    rows = (rows0, rows1, rows2, rows3)
    lsem = (lsem0, lsem1, lsem2, lsem3)
    gsem = (gsem0, gsem1, gsem2, gsem3)
    ssem = (ssem0, ssem1, ssem2, ssem3)

    def l_start(c, b):
        o = base + c * CHE
        pltpu.async_copy(src_hbm.at[pl.ds(o, CHE)], sidx[b], lsem[b])
        pltpu.async_copy(dst_hbm.at[pl.ds(o, CHE)], didx[b], lsem[b])

    def l_wait(b):
        pltpu.make_async_copy(src_hbm.at[pl.ds(base, CHE)], sidx[b],
                              lsem[b]).wait()
        pltpu.make_async_copy(dst_hbm.at[pl.ds(base, CHE)], didx[b],
                              lsem[b]).wait()

    def g_start(b):
        pltpu.async_copy(g_hbm.at[sidx[b]], rows[b], gsem[b])

    def g_wait(b):
        pltpu.make_async_copy(g_hbm.at[sidx[b]], rows[b], gsem[b]).wait()

    def s_start(b):
        pltpu.async_copy(rows[b], acc_sh.at[didx[b]], ssem[b], add=True)

    def s_wait(b):
        pltpu.make_async_copy(rows[b], acc_sh.at[didx[b]], ssem[b]).wait()

    l_start(0, 0)
    l_start(1, 1)

    def fillz(i, carry):
        for j in range(D // 16):
            rows0[i, pl.ds(j * 16, 16)] = jnp.zeros((16,), jnp.float32)
        return carry

    lax.fori_loop(0, CHE, fillz, 0)

    r0 = tid * RPT

    def zero(i, carry):
        pltpu.sync_copy(rows0, acc_sh.at[pl.ds(r0 + i * CHE, CHE)])
        return carry

    lax.fori_loop(0, RPT // CHE, zero, 0)
    plsc.subcore_barrier()

    l_wait(0)
    g_start(0)

    def visits(k, carry):
        for p in range(NB):
            bc = (1 + p) % NB
            bn = (2 + p) % NB
            bp = p
            if p >= 2:
                s_wait(bn)
            else:
                @pl.when(k >= 1)
                def _():
                    s_wait(bn)
            l_start(1 + 4 * k + p + 1, bn)
            l_wait(bc)
            g_start(bc)
            g_wait(bp)
            s_start(bp)
        return carry

    lax.fori_loop(0, 38, visits, 0)
    s_wait(2)
    l_start(154, 2)
    l_wait(1)
    g_start(1)
    g_wait(0)
    s_start(0)
    s_wait(3)
    l_start(155, 3)
    l_wait(2)
    g_start(2)
    g_wait(1)
    s_start(1)
    s_wait(0)
    l_wait(3)
    g_start(3)
    g_wait(2)
    s_start(2)
    g_wait(3)
    s_start(3)
    o = base + FCE * CHE
    pltpu.sync_copy(src_hbm.at[pl.ds(o, TAIL)], idx_t)
    pltpu.async_copy(g_hbm.at[idx_t], rows_t, tsem).wait()
    pltpu.sync_copy(dst_hbm.at[pl.ds(o, TAIL)], idx_t)
    pltpu.sync_copy(rows_t, acc_sh.at[idx_t], add=True)
    s_wait(1)
    s_wait(2)
    s_wait(3)
    plsc.subcore_barrier()
    pltpu.sync_copy(acc_sh.at[pl.ds(r0, RPT)], acc_out.at[cid, pl.ds(r0, RPT)])


_edge_call = pl.kernel(
    _edge_body,
    out_type=jax.ShapeDtypeStruct((NC, NPAD, D), jnp.float32),
    mesh=_MESH,
    scratch_types=(
        [pltpu.VMEM((CHE,), jnp.int32)] * (2 * NB)
        + [pltpu.VMEM((CHE, D), jnp.float32)] * NB
        + [
            pltpu.VMEM((TAIL,), jnp.int32),
            pltpu.VMEM((TAIL, D), jnp.float32),
            pltpu.VMEM_SHARED((NPAD, D), jnp.float32),
        ]
        + [pltpu.SemaphoreType.DMA] * (3 * NB + 1)
    ),
)


def _dense_body(x_ref, w_ref, deg_ref, g_ref):
    deg01 = deg_ref[0] + deg_ref[1]
    dis = lax.rsqrt(deg01[:, 0:1] + 1.0)
    h = jnp.dot(x_ref[:], w_ref[:], preferred_element_type=jnp.float32)
    g_ref[:] = dis * h


def _dense_call(x, w, degp):
    return pl.pallas_call(
        _dense_body,
        grid=(N // BN,),
        in_specs=[
            pl.BlockSpec((BN, D), lambda i: (i, 0)),
            pl.BlockSpec((D, D), lambda i: (0, 0)),
            pl.BlockSpec((NC, BN, DEGW), lambda i: (0, i, 0)),
        ],
        out_specs=pl.BlockSpec((BN, D), lambda i: (i, 0)),
        out_shape=jax.ShapeDtypeStruct((N, D), jnp.float32),
    )(x, w, degp)


def _final_body(acc_ref, g_ref, deg_ref, out_ref):
    deg01 = deg_ref[0] + deg_ref[1]
    dis = lax.rsqrt(deg01[:, 0:1] + 1.0)
    out_ref[:] = dis * (acc_ref[0] + acc_ref[1] + g_ref[:])


def _final_call(acc, g, degp):
    return pl.pallas_call(
        _final_body,
        grid=(N // BN,),
        in_specs=[
            pl.BlockSpec((NC, BN, D), lambda i: (0, i, 0)),
            pl.BlockSpec((BN, D), lambda i: (i, 0)),
            pl.BlockSpec((NC, BN, DEGW), lambda i: (0, i, 0)),
        ],
        out_specs=pl.BlockSpec((BN, D), lambda i: (i, 0)),
        out_shape=jax.ShapeDtypeStruct((N, D), jnp.float32),
    )(acc, g, degp)


def kernel(x, edge_index, W):
    src = edge_index[0].astype(jnp.int32)
    dst = edge_index[1].astype(jnp.int32)
    degp = _deg_call(src)
    g = _dense_call(x, W, degp)
    acc = _edge_call(g, src, dst)
    return _final_call(acc, g, degp)
